# baseline (speedup 1.0000x reference)
import jax
import jax.numpy as jnp
from jax import lax
from jax.experimental import pallas as pl
from jax.experimental.pallas import tpu as pltpu
from jax.experimental.pallas import tpu_sc as plsc

N, C, H, W = 8, 128, 224, 224
P = H * W
TOTAL = N * P
NC, NS = 2, 16
NW = NC * NS
PER_TILE = TOTAL // NW
CHUNK = 64
NCHUNK = PER_TILE // CHUNK
PAIRS = NCHUNK // 2
SUBS = CHUNK // 16
L = 16


def _sc_body(table, xs_hbm, ys_hbm, out_hbm,
             xs_v, ys_v,
             ia0, ib0, ic0, id0, wa0, wb0, wc0, wd0, ba0, bb0, bc0, bd0,
             ia1, ib1, ic1, id1, wa1, wb1, wc1, wd1, ba1, bb1, bc1, bd1,
             ob,
             sa0, sb0, sc0, sd0, sa1, sb1, sc1, sd1, so):
    setA = (ia0, ib0, ic0, id0, wa0, wb0, wc0, wd0, ba0, bb0, bc0, bd0,
            sa0, sb0, sc0, sd0)
    setB = (ia1, ib1, ic1, id1, wa1, wb1, wc1, wd1, ba1, bb1, bc1, bd1,
            sa1, sb1, sc1, sd1)

    wid = lax.axis_index("s") * NC + lax.axis_index("c")
    base_g = wid * PER_TILE
    row_base = (wid // 4) * P

    pltpu.sync_copy(xs_hbm.at[pl.ds(base_g, PER_TILE)], xs_v)
    pltpu.sync_copy(ys_hbm.at[pl.ds(base_g, PER_TILE)], ys_v)

    def fire(chk, S):
        ia, ib, ic, id_, wa_v, wb_v, wc_v, wd_v, ba, bb, bc, bd, \
            sa, sb, sc, sd = S
        off = chk * CHUNK
        for s in range(SUBS):
            xv = xs_v[pl.ds(off + s * L, L)]
            yv = ys_v[pl.ds(off + s * L, L)]
            x = ((xv + 1.0) * W - 1.0) * 0.5
            y = ((yv + 1.0) * H - 1.0) * 0.5
            xi = x.astype(jnp.int32)
            yi = y.astype(jnp.int32)
            x0 = jnp.where(xi.astype(jnp.float32) > x, xi - 1, xi)
            y0 = jnp.where(yi.astype(jnp.float32) > y, yi - 1, yi)
            x0f = x0.astype(jnp.float32)
            y0f = y0.astype(jnp.float32)
            dx1 = (x0f + 1.0) - x
            dx0 = x - x0f
            dy1 = (y0f + 1.0) - y
            dy0 = y - y0f
            one = jnp.float32(1.0)
            zero = jnp.float32(0.0)
            vx0 = jnp.where(x0 >= 0, one, zero)
            vy0 = jnp.where(y0 >= 0, one, zero)
            vx1 = jnp.where(x0 < W - 1, one, zero)
            vy1 = jnp.where(y0 < H - 1, one, zero)
            x0c = jnp.maximum(x0, 0)
            y0c = jnp.maximum(y0, 0)
            x1c = jnp.minimum(x0 + 1, W - 1)
            y1c = jnp.minimum(y0 + 1, H - 1)
            r0 = row_base + y0c * W
            r1 = row_base + y1c * W
            ia[pl.ds(s * L, L)] = r0 + x0c
            ic[pl.ds(s * L, L)] = r0 + x1c
            ib[pl.ds(s * L, L)] = r1 + x0c
            id_[pl.ds(s * L, L)] = r1 + x1c
            wa_v[pl.ds(s * L, L)] = dx1 * dy1 * (vx0 * vy0)
            wb_v[pl.ds(s * L, L)] = dx1 * dy0 * (vx0 * vy1)
            wc_v[pl.ds(s * L, L)] = dx0 * dy1 * (vx1 * vy0)
            wd_v[pl.ds(s * L, L)] = dx0 * dy0 * (vx1 * vy1)
        pltpu.async_copy(table.at[ia], ba, sa)
        pltpu.async_copy(table.at[ib], bb, sb)
        pltpu.async_copy(table.at[ic], bc, sc)
        pltpu.async_copy(table.at[id_], bd, sd)

    def wait_gathers(S):
        ia, ib, ic, id_, _, _, _, _, ba, bb, bc, bd, sa, sb, sc, sd = S
        pltpu.make_async_copy(table.at[ia], ba, sa).wait()
        pltpu.make_async_copy(table.at[ib], bb, sb).wait()
        pltpu.make_async_copy(table.at[ic], bc, sc).wait()
        pltpu.make_async_copy(table.at[id_], bd, sd).wait()

    def combine(S, half):
        _, _, _, _, wa_v, wb_v, wc_v, wd_v, ba, bb, bc, bd, \
            _, _, _, _ = S

        @plsc.parallel_loop(0, CHUNK, unroll=2)
        def p_body(p):
            pv = jnp.full((L,), 0, jnp.int32) + p
            wav = plsc.load_gather(wa_v, [pv])
            wbv = plsc.load_gather(wb_v, [pv])
            wcv = plsc.load_gather(wc_v, [pv])
            wdv = plsc.load_gather(wd_v, [pv])
            q = p + (half * CHUNK)
            for c8 in range(C // L):
                sl = pl.ds(c8 * L, L)
                ob[q, sl] = ((ba[p, sl] * wav + bb[p, sl] * wbv)
                             + (bc[p, sl] * wcv + bd[p, sl] * wdv))

    def out_copy(k):
        return pltpu.make_async_copy(
            ob, out_hbm.at[pl.ds(base_g + k * (2 * CHUNK), 2 * CHUNK)], so)

    fire(0, setA)
    out_copy(0).start()

    def step(k, carry):
        c0 = 2 * k
        fire(c0 + 1, setB)
        wait_gathers(setA)
        out_copy(k).wait()
        combine(setA, 0)
        nxt = jnp.minimum(c0 + 2, NCHUNK - 2)
        fire(nxt, setA)
        wait_gathers(setB)
        combine(setB, 1)
        out_copy(k).start()
        return carry

    lax.fori_loop(0, PAIRS, step, 0)

    out_copy(0).wait()
    wait_gathers(setA)


def _scratch_set():
    return (
        [pltpu.VMEM((CHUNK,), jnp.int32) for _ in range(4)]
        + [pltpu.VMEM((CHUNK,), jnp.float32) for _ in range(4)]
        + [pltpu.VMEM((CHUNK, C), jnp.float32) for _ in range(4)]
    )


_sc_sample = pl.kernel(
    _sc_body,
    out_type=jax.ShapeDtypeStruct((TOTAL, C), jnp.float32),
    mesh=plsc.VectorSubcoreMesh(
        core_axis_name="c", subcore_axis_name="s",
        num_cores=NC, num_subcores=NS),
    compiler_params=pltpu.CompilerParams(needs_layout_passes=False),
    scratch_types=(
        [pltpu.VMEM((PER_TILE,), jnp.float32),
         pltpu.VMEM((PER_TILE,), jnp.float32)]
        + _scratch_set()
        + _scratch_set()
        + [pltpu.VMEM((2 * CHUNK, C), jnp.float32)]
        + [pltpu.SemaphoreType.DMA] * 9
    ),
)


@jax.jit
def kernel(img, points):
    n, c, h, w = img.shape
    table = img.transpose(0, 2, 3, 1).reshape(n * h * w, c)
    xs = points[..., 0].reshape(-1)
    ys = points[..., 1].reshape(-1)
    out_t = _sc_sample(table, xs, ys)
    return out_t.reshape(n, h, w, c).transpose(0, 3, 1, 2)
